# Initial kernel scaffold; baseline (speedup 1.0000x reference)
import jax
import jax.numpy as jnp
from jax.experimental import pallas as pl


def kernel(h_V, h_E, E_idx, mask_V, mask_attend, W1, b1, W2, b2, W3, b3, W_in, b_in, W_out, b_out):
    raise NotImplementedError("write your pallas kernel here")


# speedup vs baseline: 9.8165x; 9.8165x over previous
import functools

import jax
import jax.numpy as jnp
from jax import lax
from jax.experimental import pallas as pl
from jax.experimental.pallas import tpu as pltpu
from jax.experimental.pallas import tpu_sc as plsc

BLK_N = 256
SCALE = 30.0

SC_CORES = 2
SC_SUBCORES = 16
SC_WORKERS = SC_CORES * SC_SUBCORES
GATHER_CHUNK = 128


def _gelu_tanh(x):
    return jax.nn.gelu(x, approximate=True)


def _table_kernel(hv_ref, w1g_ref, out_ref):
    out_ref[...] = jnp.dot(hv_ref[...], w1g_ref[...],
                           preferred_element_type=jnp.float32)


def _make_table(hv2, W1g):
    BN, H = hv2.shape
    return pl.pallas_call(
        _table_kernel,
        out_shape=jax.ShapeDtypeStruct((BN, H), jnp.float32),
    )(hv2, W1g)


def _sc_gather(table, flat_idx):
    E = flat_idx.shape[0]
    H = table.shape[1]
    rows_per_worker = E // SC_WORKERS
    chunks = rows_per_worker // GATHER_CHUNK
    mesh = plsc.VectorSubcoreMesh(core_axis_name="c", subcore_axis_name="s",
                                  num_cores=SC_CORES,
                                  num_subcores=SC_SUBCORES)

    @functools.partial(
        pl.kernel,
        out_type=jax.ShapeDtypeStruct((E, H), jnp.float32),
        mesh=mesh,
        scratch_types=[
            pltpu.VMEM((GATHER_CHUNK,), jnp.int32),
            pltpu.VMEM((GATHER_CHUNK, H), jnp.float32),
            pltpu.SemaphoreType.DMA,
        ],
    )
    def gather_kernel(table_hbm, idx_hbm, out_hbm, idx_v, rows_v, sem):
        wid = lax.axis_index("s") * SC_CORES + lax.axis_index("c")
        base = wid * rows_per_worker

        def body(i, carry):
            off = base + i * GATHER_CHUNK
            pltpu.sync_copy(idx_hbm.at[pl.ds(off, GATHER_CHUNK)], idx_v)
            pltpu.async_copy(table_hbm.at[idx_v], rows_v, sem).wait()
            pltpu.sync_copy(rows_v, out_hbm.at[pl.ds(off, GATHER_CHUNK)])
            return carry

        lax.fori_loop(0, chunks, body, 0)

    return gather_kernel(table, flat_idx)


def _fused_kernel(hv_ref, he_ref, gat_ref, me_ref, mv_ref,
                  w1v_ref, w1e_ref, b1_ref,
                  w2_ref, b2_ref, w3_ref, b3_ref,
                  win_ref, bin_ref, wout_ref, bout_ref,
                  out_ref, *, blk_n, k_nbr, h_dim):
    n = pl.program_id(1)
    bnk = blk_n * k_nbr

    hv_blk = hv_ref[0, pl.ds(n * blk_n, blk_n), :]
    a = jnp.dot(hv_blk, w1v_ref[...], preferred_element_type=jnp.float32)
    a = a + b1_ref[...]

    he = he_ref[0]
    e1 = jnp.dot(he, w1e_ref[...], preferred_element_type=jnp.float32)

    x = (e1 + gat_ref[0]).reshape(blk_n, k_nbr, h_dim) + a[:, None, :]
    x1 = _gelu_tanh(x).reshape(bnk, h_dim)
    x2 = _gelu_tanh(jnp.dot(x1, w2_ref[...],
                            preferred_element_type=jnp.float32) + b2_ref[...])

    me = me_ref[0]
    x2m = (x2 * me).reshape(blk_n, k_nbr, h_dim)
    s = jnp.sum(x2m, axis=1)
    cnt = jnp.sum(me.reshape(blk_n, k_nbr, 1), axis=1)
    dh = (jnp.dot(s, w3_ref[...], preferred_element_type=jnp.float32)
          + cnt * b3_ref[...]) * (1.0 / SCALE)

    h = hv_blk + dh
    z = jnp.dot(h, win_ref[...],
                preferred_element_type=jnp.float32) + bin_ref[...]
    u = z * 0.5 * (1.0 + lax.erf(z * (2.0 ** -0.5)))
    y = jnp.dot(u, wout_ref[...], preferred_element_type=jnp.float32)
    y = y + bout_ref[...]
    out_ref[0] = mv_ref[0] * (h + y)


def kernel(h_V, h_E, E_idx, mask_V, mask_attend, W1, b1, W2, b2, W3, b3,
           W_in, b_in, W_out, b_out):
    B, N, K, H = h_E.shape
    blk_n = min(BLK_N, N)
    bnk = blk_n * K
    grid = (B, N // blk_n)

    W1v, W1e, W1g = W1[:H], W1[H:2 * H], W1[2 * H:]

    g_table = _make_table(h_V.reshape(B * N, H), W1g)

    flat_idx = (E_idx.reshape(B, N * K)
                + (jnp.arange(B, dtype=jnp.int32) * N)[:, None]).reshape(-1)
    gathered = _sc_gather(g_table, flat_idx).reshape(B, N * K, H)

    hE2 = h_E.reshape(B, N * K, H)
    maskE = mask_attend.reshape(B, N * K, 1)
    maskV2 = mask_V.reshape(B, N, 1)

    def row(v):
        return v.reshape(1, -1)

    kern = functools.partial(_fused_kernel, blk_n=blk_n, k_nbr=K, h_dim=H)

    out = pl.pallas_call(
        kern,
        grid=grid,
        in_specs=[
            pl.BlockSpec((1, N, H), lambda b, n: (b, 0, 0)),
            pl.BlockSpec((1, bnk, H), lambda b, n: (b, n, 0)),
            pl.BlockSpec((1, bnk, H), lambda b, n: (b, n, 0)),
            pl.BlockSpec((1, bnk, 1), lambda b, n: (b, n, 0)),
            pl.BlockSpec((1, blk_n, 1), lambda b, n: (b, n, 0)),
            pl.BlockSpec((H, H), lambda b, n: (0, 0)),
            pl.BlockSpec((H, H), lambda b, n: (0, 0)),
            pl.BlockSpec((1, H), lambda b, n: (0, 0)),
            pl.BlockSpec((H, H), lambda b, n: (0, 0)),
            pl.BlockSpec((1, H), lambda b, n: (0, 0)),
            pl.BlockSpec((H, H), lambda b, n: (0, 0)),
            pl.BlockSpec((1, H), lambda b, n: (0, 0)),
            pl.BlockSpec((H, 4 * H), lambda b, n: (0, 0)),
            pl.BlockSpec((1, 4 * H), lambda b, n: (0, 0)),
            pl.BlockSpec((4 * H, H), lambda b, n: (0, 0)),
            pl.BlockSpec((1, H), lambda b, n: (0, 0)),
        ],
        out_specs=pl.BlockSpec((1, blk_n, H), lambda b, n: (b, n, 0)),
        out_shape=jax.ShapeDtypeStruct((B, N, H), jnp.float32),
        compiler_params=pltpu.CompilerParams(
            dimension_semantics=("arbitrary", "arbitrary"),
        ),
    )(h_V, hE2, gathered, maskE, maskV2,
      W1v, W1e, row(b1), W2, row(b2), W3, row(b3),
      W_in, row(b_in), W_out, row(b_out))
    return out


# speedup vs baseline: 11.1920x; 1.1401x over previous
import functools

import jax
import jax.numpy as jnp
from jax import lax
from jax.experimental import pallas as pl
from jax.experimental.pallas import tpu as pltpu
from jax.experimental.pallas import tpu_sc as plsc

BLK_N = 256
SCALE = 30.0

SC_CORES = 2
SC_SUBCORES = 16
SC_WORKERS = SC_CORES * SC_SUBCORES
GATHER_CHUNK = 128


def _gelu_tanh(x):
    return jax.nn.gelu(x, approximate=True)


def _table_kernel(hv_ref, w1g_ref, out_ref):
    out_ref[...] = jnp.dot(hv_ref[...], w1g_ref[...],
                           preferred_element_type=jnp.float32)


def _make_table(hv2, W1g):
    BN, H = hv2.shape
    return pl.pallas_call(
        _table_kernel,
        out_shape=jax.ShapeDtypeStruct((BN, H), jnp.float32),
    )(hv2, W1g)


NBUF = 4


def _sc_gather(table, flat_idx):
    E = flat_idx.shape[0]
    H = table.shape[1]
    rows_per_worker = E // SC_WORKERS
    chunks = rows_per_worker // GATHER_CHUNK
    groups = chunks // NBUF
    mesh = plsc.VectorSubcoreMesh(core_axis_name="c", subcore_axis_name="s",
                                  num_cores=SC_CORES,
                                  num_subcores=SC_SUBCORES)

    @functools.partial(
        pl.kernel,
        out_type=jax.ShapeDtypeStruct((E, H), jnp.float32),
        mesh=mesh,
        scratch_types=[
            pltpu.VMEM((rows_per_worker,), jnp.int32),
            [pltpu.VMEM((GATHER_CHUNK, H), jnp.float32)] * NBUF,
            [pltpu.SemaphoreType.DMA] * NBUF,
        ],
    )
    def gather_kernel(table_hbm, idx_hbm, out_hbm, idx_v, rows, sems):
        wid = lax.axis_index("s") * SC_CORES + lax.axis_index("c")
        base = wid * rows_per_worker
        pltpu.sync_copy(idx_hbm.at[pl.ds(base, rows_per_worker)], idx_v)

        def start(chunk, b):
            idx_slice = idx_v.at[pl.ds(chunk * GATHER_CHUNK, GATHER_CHUNK)]
            return pltpu.async_copy(table_hbm.at[idx_slice], rows[b], sems[b])

        for b in range(NBUF):
            start(b, b)

        def body(j, carry):
            for b in range(NBUF):
                chunk = j * NBUF + b
                pltpu.make_async_copy(table_hbm.at[idx_v.at[
                    pl.ds(chunk * GATHER_CHUNK, GATHER_CHUNK)]],
                    rows[b], sems[b]).wait()
                pltpu.sync_copy(
                    rows[b],
                    out_hbm.at[pl.ds(base + chunk * GATHER_CHUNK,
                                     GATHER_CHUNK)])

                @pl.when(j < groups - 1)
                def _():
                    start((j + 1) * NBUF + b, b)
            return carry

        lax.fori_loop(0, groups, body, 0)

    return gather_kernel(table, flat_idx)


def _fused_kernel(hv_ref, he_ref, gat_ref, me_ref, mv_ref,
                  w1v_ref, w1e_ref, b1_ref,
                  w2_ref, b2_ref, w3_ref, b3_ref,
                  win_ref, bin_ref, wout_ref, bout_ref,
                  out_ref, *, blk_n, k_nbr, h_dim):
    n = pl.program_id(1)
    bnk = blk_n * k_nbr

    hv_blk = hv_ref[0, pl.ds(n * blk_n, blk_n), :]
    a = jnp.dot(hv_blk, w1v_ref[...], preferred_element_type=jnp.float32)
    a = a + b1_ref[...]

    he = he_ref[0]
    e1 = jnp.dot(he, w1e_ref[...], preferred_element_type=jnp.float32)

    x = (e1 + gat_ref[0]).reshape(blk_n, k_nbr, h_dim) + a[:, None, :]
    x1 = _gelu_tanh(x).reshape(bnk, h_dim)
    x2 = _gelu_tanh(jnp.dot(x1, w2_ref[...],
                            preferred_element_type=jnp.float32) + b2_ref[...])

    me = me_ref[0]
    x2m = (x2 * me).reshape(blk_n, k_nbr, h_dim)
    s = jnp.sum(x2m, axis=1)
    cnt = jnp.sum(me.reshape(blk_n, k_nbr, 1), axis=1)
    dh = (jnp.dot(s, w3_ref[...], preferred_element_type=jnp.float32)
          + cnt * b3_ref[...]) * (1.0 / SCALE)

    h = hv_blk + dh
    z = jnp.dot(h, win_ref[...],
                preferred_element_type=jnp.float32) + bin_ref[...]
    u = z * 0.5 * (1.0 + lax.erf(z * (2.0 ** -0.5)))
    y = jnp.dot(u, wout_ref[...], preferred_element_type=jnp.float32)
    y = y + bout_ref[...]
    out_ref[0] = mv_ref[0] * (h + y)


def kernel(h_V, h_E, E_idx, mask_V, mask_attend, W1, b1, W2, b2, W3, b3,
           W_in, b_in, W_out, b_out):
    B, N, K, H = h_E.shape
    blk_n = min(BLK_N, N)
    bnk = blk_n * K
    grid = (B, N // blk_n)

    W1v, W1e, W1g = W1[:H], W1[H:2 * H], W1[2 * H:]

    g_table = _make_table(h_V.reshape(B * N, H), W1g)

    flat_idx = (E_idx.reshape(B, N * K)
                + (jnp.arange(B, dtype=jnp.int32) * N)[:, None]).reshape(-1)
    gathered = _sc_gather(g_table, flat_idx).reshape(B, N * K, H)

    hE2 = h_E.reshape(B, N * K, H)
    maskE = mask_attend.reshape(B, N * K, 1)
    maskV2 = mask_V.reshape(B, N, 1)

    def row(v):
        return v.reshape(1, -1)

    kern = functools.partial(_fused_kernel, blk_n=blk_n, k_nbr=K, h_dim=H)

    out = pl.pallas_call(
        kern,
        grid=grid,
        in_specs=[
            pl.BlockSpec((1, N, H), lambda b, n: (b, 0, 0)),
            pl.BlockSpec((1, bnk, H), lambda b, n: (b, n, 0)),
            pl.BlockSpec((1, bnk, H), lambda b, n: (b, n, 0)),
            pl.BlockSpec((1, bnk, 1), lambda b, n: (b, n, 0)),
            pl.BlockSpec((1, blk_n, 1), lambda b, n: (b, n, 0)),
            pl.BlockSpec((H, H), lambda b, n: (0, 0)),
            pl.BlockSpec((H, H), lambda b, n: (0, 0)),
            pl.BlockSpec((1, H), lambda b, n: (0, 0)),
            pl.BlockSpec((H, H), lambda b, n: (0, 0)),
            pl.BlockSpec((1, H), lambda b, n: (0, 0)),
            pl.BlockSpec((H, H), lambda b, n: (0, 0)),
            pl.BlockSpec((1, H), lambda b, n: (0, 0)),
            pl.BlockSpec((H, 4 * H), lambda b, n: (0, 0)),
            pl.BlockSpec((1, 4 * H), lambda b, n: (0, 0)),
            pl.BlockSpec((4 * H, H), lambda b, n: (0, 0)),
            pl.BlockSpec((1, H), lambda b, n: (0, 0)),
        ],
        out_specs=pl.BlockSpec((1, blk_n, H), lambda b, n: (b, n, 0)),
        out_shape=jax.ShapeDtypeStruct((B, N, H), jnp.float32),
        compiler_params=pltpu.CompilerParams(
            dimension_semantics=("arbitrary", "arbitrary"),
        ),
    )(h_V, hE2, gathered, maskE, maskV2,
      W1v, W1e, row(b1), W2, row(b2), W3, row(b3),
      W_in, row(b_in), W_out, row(b_out))
    return out
